# Initial kernel scaffold; baseline (speedup 1.0000x reference)
import jax
import jax.numpy as jnp
from jax.experimental import pallas as pl


def kernel(x, W_enc, b_enc, W_dec, b_dec):
    raise NotImplementedError("write your pallas kernel here")


# speedup vs baseline: 15.8019x; 15.8019x over previous
import functools

import jax
import jax.numpy as jnp
from jax.experimental import pallas as pl

_K = 64
_BISECT_ITERS = 26


def _sae_block_kernel(x_ref, we_ref, benc_ref, wd_ref, bdec_ref,
                      recon_ref, xh_ref, z_ref):
    xb = x_ref[...]
    xc = xb - bdec_ref[...]
    pre = jax.lax.dot_general(
        xc.astype(jnp.bfloat16), we_ref[...],
        (((1,), (0,)), ((), ())),
        preferred_element_type=jnp.float32,
    ) + benc_ref[...]

    lo0 = jnp.min(pre, axis=1, keepdims=True)
    hi0 = jnp.max(pre, axis=1, keepdims=True)

    def body(_, carry):
        lo, hi = carry
        mid = 0.5 * (lo + hi)
        cnt = jnp.sum((pre >= mid).astype(jnp.float32), axis=1, keepdims=True)
        ge = cnt >= float(_K)
        return jnp.where(ge, mid, lo), jnp.where(ge, hi, mid)

    thr, _ = jax.lax.fori_loop(0, _BISECT_ITERS, body, (lo0, hi0))

    zb = jnp.where(pre >= thr, jnp.maximum(pre, 0.0), 0.0)
    z_ref[...] = zb

    xh = jax.lax.dot_general(
        zb.astype(jnp.bfloat16), wd_ref[...],
        (((1,), (0,)), ((), ())),
        preferred_element_type=jnp.float32,
    ) + bdec_ref[...]
    xh_ref[...] = xh

    d = xh - xb
    part = jnp.sum(d * d)
    prev = jnp.where(pl.program_id(0) == 0, 0.0, recon_ref[...])
    recon_ref[...] = prev + part


@functools.partial(jax.jit, static_argnames=())
def kernel(x, W_enc, b_enc, W_dec, b_dec):
    B, T, D = x.shape
    S = W_enc.shape[0]
    N = B * T
    xf = x.reshape(N, D)
    we = W_enc.T.astype(jnp.bfloat16)
    wd = W_dec.T.astype(jnp.bfloat16)
    benc = b_enc.reshape(1, S)
    bdec = b_dec.reshape(1, D)

    rb = 128 if N % 128 == 0 else N
    grid = N // rb

    recon_sum, xh, z = pl.pallas_call(
        _sae_block_kernel,
        grid=(grid,),
        in_specs=[
            pl.BlockSpec((rb, D), lambda i: (i, 0)),
            pl.BlockSpec((D, S), lambda i: (0, 0)),
            pl.BlockSpec((1, S), lambda i: (0, 0)),
            pl.BlockSpec((S, D), lambda i: (0, 0)),
            pl.BlockSpec((1, D), lambda i: (0, 0)),
        ],
        out_specs=[
            pl.BlockSpec((1, 1), lambda i: (0, 0)),
            pl.BlockSpec((rb, D), lambda i: (i, 0)),
            pl.BlockSpec((rb, S), lambda i: (i, 0)),
        ],
        out_shape=[
            jax.ShapeDtypeStruct((1, 1), jnp.float32),
            jax.ShapeDtypeStruct((N, D), jnp.float32),
            jax.ShapeDtypeStruct((N, S), jnp.float32),
        ],
    )(xf, we, benc, wd, bdec)

    recon = recon_sum[0, 0] / N
    return recon, xh.reshape(B, T, D), z.reshape(B, T, S)


# speedup vs baseline: 17.6248x; 1.1154x over previous
import functools

import jax
import jax.numpy as jnp
from jax.experimental import pallas as pl

_K = 64
_SEARCH_ITERS = 20


def _topk_threshold(pre):
    n = pre.shape[1]
    lo = jnp.min(pre, axis=1, keepdims=True)
    hi = jnp.max(pre, axis=1, keepdims=True)
    flo = jnp.full_like(lo, float(n) - (_K - 0.5))
    fhi = jnp.full_like(lo, 1.0 - (_K - 0.5))
    side = jnp.zeros_like(lo)

    def body(it, carry):
        lo, hi, flo, fhi, side = carry
        w = hi - lo
        t_fp = lo + w * (flo / (flo - fhi))
        t_bis = lo + 0.5 * w
        t = jnp.where(it < 2, t_bis,
                      jnp.clip(t_fp, lo + 0.01 * w, hi - 0.01 * w))
        cnt = jnp.sum((pre >= t).astype(jnp.float32), axis=1, keepdims=True)
        f = cnt - (_K - 0.5)
        up = f > 0.0
        nlo = jnp.where(up, t, lo)
        nhi = jnp.where(up, hi, t)
        nflo = jnp.where(up, f, jnp.where(side < 0.0, 0.5 * flo, flo))
        nfhi = jnp.where(up, jnp.where(side > 0.0, 0.5 * fhi, fhi), f)
        nside = jnp.where(up, 1.0, -1.0)
        return nlo, nhi, nflo, nfhi, nside

    lo, _, _, _, _ = jax.lax.fori_loop(0, _SEARCH_ITERS, body,
                                       (lo, hi, flo, fhi, side))
    return lo


def _sae_block_kernel(x_ref, we_ref, benc_ref, wd_ref, bdec_ref,
                      recon_ref, xh_ref, z_ref):
    xb = x_ref[...]
    xc = xb - bdec_ref[...]
    pre = jax.lax.dot_general(
        xc.astype(jnp.bfloat16), we_ref[...],
        (((1,), (0,)), ((), ())),
        preferred_element_type=jnp.float32,
    ) + benc_ref[...]

    thr = _topk_threshold(pre)

    zb = jnp.where(pre >= thr, jnp.maximum(pre, 0.0), 0.0)
    z_ref[...] = zb

    xh = jax.lax.dot_general(
        zb.astype(jnp.bfloat16), wd_ref[...],
        (((1,), (0,)), ((), ())),
        preferred_element_type=jnp.float32,
    ) + bdec_ref[...]
    xh_ref[...] = xh

    d = xh - xb
    part = jnp.sum(d * d)
    prev = jnp.where(pl.program_id(0) == 0, 0.0, recon_ref[...])
    recon_ref[...] = prev + part


@functools.partial(jax.jit, static_argnames=())
def kernel(x, W_enc, b_enc, W_dec, b_dec):
    B, T, D = x.shape
    S = W_enc.shape[0]
    N = B * T
    xf = x.reshape(N, D)
    we = W_enc.T.astype(jnp.bfloat16)
    wd = W_dec.T.astype(jnp.bfloat16)
    benc = b_enc.reshape(1, S)
    bdec = b_dec.reshape(1, D)

    rb = 128 if N % 128 == 0 else N
    grid = N // rb

    recon_sum, xh, z = pl.pallas_call(
        _sae_block_kernel,
        grid=(grid,),
        in_specs=[
            pl.BlockSpec((rb, D), lambda i: (i, 0)),
            pl.BlockSpec((D, S), lambda i: (0, 0)),
            pl.BlockSpec((1, S), lambda i: (0, 0)),
            pl.BlockSpec((S, D), lambda i: (0, 0)),
            pl.BlockSpec((1, D), lambda i: (0, 0)),
        ],
        out_specs=[
            pl.BlockSpec((1, 1), lambda i: (0, 0)),
            pl.BlockSpec((rb, D), lambda i: (i, 0)),
            pl.BlockSpec((rb, S), lambda i: (i, 0)),
        ],
        out_shape=[
            jax.ShapeDtypeStruct((1, 1), jnp.float32),
            jax.ShapeDtypeStruct((N, D), jnp.float32),
            jax.ShapeDtypeStruct((N, S), jnp.float32),
        ],
    )(xf, we, benc, wd, bdec)

    recon = recon_sum[0, 0] / N
    return recon, xh.reshape(B, T, D), z.reshape(B, T, S)
